# Initial kernel scaffold; baseline (speedup 1.0000x reference)
import jax
import jax.numpy as jnp
from jax.experimental import pallas as pl


def kernel(x, w, src_idx, dst_idx):
    raise NotImplementedError("write your pallas kernel here")


# speedup vs baseline: 40.5582x; 40.5582x over previous
import functools

import jax
import jax.numpy as jnp
from jax import lax
from jax.experimental import pallas as pl
from jax.experimental.pallas import tpu as pltpu
from jax.experimental.pallas import tpu_sc as plsc

NSEG = 4
EXTENT = 32
F = NSEG * EXTENT

NC = 2
NS = 16
NW = NC * NS

CH = 128
LANES = 16


def _sc_call(n_pad, e_pad):
    ew = e_pad // NW
    nch = ew // CH
    rows_per_tile = n_pad // NS
    nz = rows_per_tile // CH

    mesh = plsc.VectorSubcoreMesh(core_axis_name="c", subcore_axis_name="s")

    @functools.partial(
        pl.kernel,
        out_type=jax.ShapeDtypeStruct((NC, n_pad, F), jnp.float32),
        mesh=mesh,
        scratch_types=[
            pltpu.MemorySpace.VMEM_SHARED((n_pad, F), jnp.float32),
            pltpu.MemorySpace.VMEM((CH,), jnp.int32),
            pltpu.MemorySpace.VMEM((CH,), jnp.int32),
            pltpu.MemorySpace.VMEM((CH * NSEG,), jnp.float32),
            pltpu.MemorySpace.VMEM((CH, F), jnp.float32),
            pltpu.SemaphoreType.DMA,
        ],
    )
    def k(x_hbm, s_hbm, d_hbm, w_hbm, out_hbm,
          acc, sidx_v, didx_v, w_v, rows_v, sem):
        cid = lax.axis_index("c")
        sid = lax.axis_index("s")
        wid = cid * NS + sid

        def zero_row(r, carry):
            for h in range(F // LANES):
                rows_v[r, pl.ds(h * LANES, LANES)] = jnp.zeros(
                    (LANES,), jnp.float32)
            return carry
        lax.fori_loop(0, CH, zero_row, 0)
        for i in range(nz):
            pltpu.sync_copy(rows_v,
                            acc.at[pl.ds(sid * rows_per_tile + i * CH, CH)])
        plsc.subcore_barrier()

        def chunk(c, carry):
            ebase = wid * ew + c * CH
            pltpu.sync_copy(s_hbm.at[pl.ds(ebase, CH)], sidx_v)
            pltpu.sync_copy(d_hbm.at[pl.ds(ebase, CH)], didx_v)
            pltpu.sync_copy(w_hbm.at[pl.ds(ebase * NSEG, CH * NSEG)], w_v)

            pltpu.async_copy(x_hbm.at[sidx_v], rows_v, sem).wait()

            def mul(g, carry2):
                wvec = w_v[pl.ds(g * LANES, LANES)]
                for q in range(4):
                    for s in range(NSEG):
                        ws = wvec[q * NSEG + s]
                        for h in range(EXTENT // LANES):
                            off = s * EXTENT + h * LANES
                            rows_v[g * 4 + q, pl.ds(off, LANES)] = (
                                rows_v[g * 4 + q, pl.ds(off, LANES)] * ws)
                return carry2
            lax.fori_loop(0, CH // 4, mul, 0)

            pltpu.sync_copy(rows_v, acc.at[didx_v], add=True)
            return carry
        lax.fori_loop(0, nch, chunk, 0)

        plsc.subcore_barrier()

        for i in range(nz):
            base = sid * rows_per_tile + i * CH
            pltpu.sync_copy(acc.at[pl.ds(base, CH)],
                            out_hbm.at[cid, pl.ds(base, CH)])

    return k


def kernel(x, w, src_idx, dst_idx):
    n_nodes, f = x.shape
    e = w.shape[0]
    grain = NW * CH
    e_pad = ((e + grain - 1) // grain) * grain
    pad = e_pad - e
    if pad:
        src_idx = jnp.concatenate(
            [src_idx, jnp.zeros((pad,), jnp.int32)])
        dst_idx = jnp.concatenate(
            [dst_idx, jnp.zeros((pad,), jnp.int32)])
        w = jnp.concatenate([w, jnp.zeros((pad, NSEG), jnp.float32)])

    ngrain = NS * CH
    n_pad = ((n_nodes + ngrain - 1) // ngrain) * ngrain

    partials = _sc_call(n_pad, e_pad)(x, src_idx, dst_idx, w.reshape(-1))
    return partials[0, :n_nodes] + partials[1, :n_nodes]


# speedup vs baseline: 61.4804x; 1.5159x over previous
import functools

import jax
import jax.numpy as jnp
from jax import lax
from jax.experimental import pallas as pl
from jax.experimental.pallas import tpu as pltpu
from jax.experimental.pallas import tpu_sc as plsc

NSEG = 4
EXTENT = 32
F = NSEG * EXTENT

NC = 2
NS = 16
NW = NC * NS

CH = 128
LANES = 16


def _sc_call(n_nodes, n_pad, e):
    ew = e // NW
    assert e % NW == 0 and ew % 8 == 0
    nch = ew // CH
    tail = ew - nch * CH
    assert tail % LANES == 0
    rows_per_tile = n_pad // NS
    nz = rows_per_tile // CH

    mesh = plsc.VectorSubcoreMesh(core_axis_name="c", subcore_axis_name="s")

    @functools.partial(
        pl.kernel,
        out_type=jax.ShapeDtypeStruct((NC, n_pad, F), jnp.float32),
        mesh=mesh,
        scratch_types=[
            pltpu.MemorySpace.VMEM_SHARED((n_pad, F), jnp.float32),
            pltpu.MemorySpace.VMEM((CH,), jnp.int32),
            pltpu.MemorySpace.VMEM((CH,), jnp.int32),
            pltpu.MemorySpace.VMEM((max(tail, 1),), jnp.int32),
            pltpu.MemorySpace.VMEM((CH * NSEG,), jnp.float32),
            pltpu.MemorySpace.VMEM((CH, F), jnp.float32),
            pltpu.SemaphoreType.DMA,
        ],
    )
    def k(x_hbm, s_hbm, d_hbm, w_hbm, out_hbm,
          acc, sidx_v, didx_v, didx_t, wq_v, rows_v, sem):
        cid = lax.axis_index("c")
        sid = lax.axis_index("s")
        wid = cid * NS + sid

        def zero_row(r, carry):
            for h in range(F // LANES):
                rows_v[r, pl.ds(h * LANES, LANES)] = jnp.zeros(
                    (LANES,), jnp.float32)
            return carry
        lax.fori_loop(0, CH, zero_row, 0)
        for i in range(nz):
            pltpu.sync_copy(rows_v,
                            acc.at[pl.ds(sid * rows_per_tile + i * CH, CH)])
        plsc.subcore_barrier()

        def do_edges(ebase, n, d_ref):
            s_ref = sidx_v.at[pl.ds(0, n)] if n != CH else sidx_v
            pltpu.sync_copy(s_hbm.at[pl.ds(ebase, n)], s_ref)
            pltpu.sync_copy(d_hbm.at[pl.ds(ebase, n)], d_ref)
            pltpu.sync_copy(w_hbm.at[pl.ds(ebase * NSEG, n * NSEG)],
                            wq_v.at[pl.ds(0, n * NSEG)])
            pltpu.async_copy(
                x_hbm.at[s_ref], rows_v.at[pl.ds(0, n)], sem).wait()

            def mul(g, carry2):
                wvec = wq_v[pl.ds(g * LANES, LANES)]
                for q in range(4):
                    for s in range(NSEG):
                        ws = wvec[q * NSEG + s]
                        for h in range(EXTENT // LANES):
                            off = s * EXTENT + h * LANES
                            rows_v[g * 4 + q, pl.ds(off, LANES)] = (
                                rows_v[g * 4 + q, pl.ds(off, LANES)] * ws)
                return carry2
            lax.fori_loop(0, n // 4, mul, 0)

            pltpu.sync_copy(rows_v.at[pl.ds(0, n)], acc.at[d_ref], add=True)

        def chunk(c, carry):
            do_edges(wid * ew + c * CH, CH, didx_v)
            return carry
        lax.fori_loop(0, nch, chunk, 0)
        if tail:
            do_edges(wid * ew + nch * CH, tail, didx_t)

        plsc.subcore_barrier()

        for i in range(nz):
            base = sid * rows_per_tile + i * CH
            pltpu.sync_copy(acc.at[pl.ds(base, CH)],
                            out_hbm.at[cid, pl.ds(base, CH)])

    return k


def kernel(x, w, src_idx, dst_idx):
    n_nodes, f = x.shape
    e = w.shape[0]
    ngrain = NS * CH
    n_pad = ((n_nodes + ngrain - 1) // ngrain) * ngrain

    partials = _sc_call(n_nodes, n_pad, e)(x, src_idx, dst_idx,
                                           w.reshape(-1))
    return partials[0, :n_nodes] + partials[1, :n_nodes]
